# Initial kernel scaffold; baseline (speedup 1.0000x reference)
import jax
import jax.numpy as jnp
from jax.experimental import pallas as pl


def kernel(x, W_l0, b_l0, W_r0, W_l1, b_l1, W_r1, edge_index0, edge_index1):
    raise NotImplementedError("write your pallas kernel here")


# speedup vs baseline: 4.7145x; 4.7145x over previous
import functools

import jax
import jax.numpy as jnp
from jax import lax
from jax.experimental import pallas as pl
from jax.experimental.pallas import tpu as pltpu
from jax.experimental.pallas import tpu_sc as plsc

N0, N1, N2 = 10000, 2000, 400
D = 128
DE = 144
CH = 128
NW = 32


def _make_agg(e_pad, n_dst):
  n_chunks = e_pad // CH
  assert n_chunks % NW == 0
  chunks_per_w = n_chunks // NW
  n_acc = ((n_dst + 1 + 127) // 128) * 128
  rows_init = n_acc // 16
  mesh = plsc.VectorSubcoreMesh(core_axis_name="c", subcore_axis_name="s")

  @functools.partial(
      pl.kernel,
      out_type=jax.ShapeDtypeStruct((2, n_acc, DE), jnp.float32),
      mesh=mesh,
      compiler_params=pltpu.CompilerParams(use_tc_tiling_on_sc=False),
      scratch_types=[
          pltpu.VMEM((1, CH), jnp.int32),
          pltpu.VMEM((1, CH), jnp.int32),
          pltpu.VMEM((CH, DE), jnp.float32),
          pltpu.VMEM_SHARED((n_acc, DE), jnp.float32),
          pltpu.SemaphoreType.DMA,
      ],
  )
  def agg(xext, src, dst, zeros, out, src_v, dst_v, rows_v, acc, sem):
    c = lax.axis_index("c")
    s = lax.axis_index("s")
    wid = s * 2 + c
    pltpu.sync_copy(zeros.at[pl.ds(s * rows_init, rows_init)],
                    acc.at[pl.ds(s * rows_init, rows_init)])
    plsc.subcore_barrier()

    def body(j, carry):
      base = (wid + NW * j) * CH
      pltpu.sync_copy(src.at[pl.ds(base, CH)], src_v.at[0])
      pltpu.sync_copy(dst.at[pl.ds(base, CH)], dst_v.at[0])
      pltpu.async_copy(xext.at[src_v.at[0]], rows_v, sem).wait()
      pltpu.sync_copy(rows_v, acc.at[dst_v.at[0]], add=True)
      return carry

    lax.fori_loop(0, chunks_per_w, body, 0)
    plsc.subcore_barrier()
    pltpu.sync_copy(acc.at[pl.ds(s * rows_init, rows_init)],
                    out.at[c, pl.ds(s * rows_init, rows_init)])

  return agg


def _make_post(n_tgt, relu, make_ext):
  out_cols = DE if make_ext else D

  def body(accp_ref, xt_ref, wl_ref, bl_ref, wr_ref, out_ref):
    acc = accp_ref[0] + accp_ref[1]
    acc = acc[:n_tgt]
    ssum = acc[:, :D]
    deg = acc[:, D:D + 1]
    mean = ssum / jnp.maximum(deg, 1.0)
    h = lax.dot_general(mean, wl_ref[...], (((1,), (1,)), ((), ())),
                        preferred_element_type=jnp.float32)
    h = h + bl_ref[...]
    h = h + lax.dot_general(xt_ref[...], wr_ref[...], (((1,), (1,)), ((), ())),
                            preferred_element_type=jnp.float32)
    if relu:
      h = jnp.maximum(h, 0.0)
    if make_ext:
      h = jnp.concatenate(
          [h, jnp.ones((n_tgt, DE - D), jnp.float32)], axis=1)
    out_ref[...] = h

  return pl.pallas_call(
      body,
      out_shape=jax.ShapeDtypeStruct((n_tgt, out_cols), jnp.float32),
  )


def _pad_edges(src, dst, n_dst):
  e = src.shape[0]
  npad = _epad(e) - e
  src_p = jnp.concatenate([src, jnp.zeros((npad,), jnp.int32)])
  dst_p = jnp.concatenate([dst, jnp.full((npad,), n_dst, jnp.int32)])
  return src_p, dst_p


def _epad(e):
  return ((e + CH * NW - 1) // (CH * NW)) * (CH * NW)


_agg0 = _make_agg(_epad(320000), N1)
_agg1 = _make_agg(_epad(64000), N2)
_post0 = _make_post(N1, relu=True, make_ext=True)
_post1 = _make_post(N2, relu=False, make_ext=False)


def kernel(x, W_l0, b_l0, W_r0, W_l1, b_l1, W_r1, edge_index0, edge_index1):
  x = x.astype(jnp.float32)
  xext = jnp.concatenate([x, jnp.ones((N0, DE - D), jnp.float32)], axis=1)
  src0 = edge_index0[0].astype(jnp.int32)
  dst0 = edge_index0[1].astype(jnp.int32)
  src1 = edge_index1[0].astype(jnp.int32)
  dst1 = edge_index1[1].astype(jnp.int32)
  src0, dst0 = _pad_edges(src0, dst0, N1)
  src1, dst1 = _pad_edges(src1, dst1, N2)

  zeros0 = jnp.zeros((2048, DE), jnp.float32)
  zeros1 = jnp.zeros((512, DE), jnp.float32)

  acc0 = _agg0(xext, src0, dst0, zeros0)
  hext = _post0(acc0, x[:N1], W_l0, b_l0.reshape(1, D), W_r0)
  acc1 = _agg1(hext, src1, dst1, zeros1)
  out = _post1(acc1, hext[:N2, :D], W_l1, b_l1.reshape(1, D), W_r1)
  return out
